# Initial kernel scaffold; baseline (speedup 1.0000x reference)
import jax
import jax.numpy as jnp
from jax.experimental import pallas as pl


def kernel(x, edge_index, edge_weight, W):
    raise NotImplementedError("write your pallas kernel here")


# speedup vs baseline: 3.2298x; 3.2298x over previous
import jax
import jax.numpy as jnp
from jax import lax
from jax.experimental import pallas as pl
from jax.experimental.pallas import tpu as pltpu
from jax.experimental.pallas import tpu_sc as plsc

_NC = 2
_NS = 16
_NW = _NC * _NS
_L = 16

_K = 10
_ALPHA = 0.5


def _mesh():
    return plsc.VectorSubcoreMesh(core_axis_name="c", subcore_axis_name="s")


def _wid():
    return lax.axis_index("s") * _NC + lax.axis_index("c")


def _make_deg_pack(N, E):
    ept = E // _NW

    def body(row_hbm, col_hbm, ew_hbm, parts_hbm, packed_hbm, rowb, colb, wb, degb, pkb):
        w = _wid()
        base = w * ept
        pltpu.sync_copy(row_hbm.at[pl.ds(base, ept)], rowb)
        pltpu.sync_copy(col_hbm.at[pl.ds(base, ept)], colb)
        pltpu.sync_copy(ew_hbm.at[pl.ds(base, ept)], wb)

        zeros = jnp.zeros((_L,), jnp.float32)

        @pl.loop(0, N // _L)
        def _zero(i):
            degb[pl.ds(i * _L, _L)] = zeros

        @pl.loop(0, ept // _L)
        def _edges(i):
            sl = pl.ds(i * _L, _L)
            r = rowb[sl]
            c = colb[sl]
            wv = wb[sl]
            plsc.addupdate_scatter(degb, [c], wv)
            pkb[sl] = r | (c << 16)

        pltpu.sync_copy(degb, parts_hbm.at[pl.ds(w * N, N)])
        pltpu.sync_copy(pkb, packed_hbm.at[pl.ds(base, ept)])

    return pl.kernel(
        body,
        out_type=(
            jax.ShapeDtypeStruct((_NW * N,), jnp.float32),
            jax.ShapeDtypeStruct((E,), jnp.int32),
        ),
        mesh=_mesh(),
        compiler_params=pltpu.CompilerParams(needs_layout_passes=False),
        scratch_types=[
            pltpu.VMEM((ept,), jnp.int32),
            pltpu.VMEM((ept,), jnp.int32),
            pltpu.VMEM((ept,), jnp.float32),
            pltpu.VMEM((N,), jnp.float32),
            pltpu.VMEM((ept,), jnp.int32),
        ],
    )


def _tc_prep(x, W, parts):
    N, _ = x.shape
    D = W.shape[1]

    def body(x_ref, w_ref, parts_ref, h0t_ref, dis_ref):
        h0 = jnp.dot(x_ref[...], w_ref[...], preferred_element_type=jnp.float32)
        h0t_ref[...] = h0.T
        deg = jnp.sum(parts_ref[...], axis=0)
        dis = jnp.where(deg > 0, lax.rsqrt(deg), 0.0)
        dis_ref[...] = dis.reshape(1, N)

    return pl.pallas_call(
        body,
        out_shape=(
            jax.ShapeDtypeStruct((D, N), jnp.float32),
            jax.ShapeDtypeStruct((1, N), jnp.float32),
        ),
    )(x, W, parts)


def _make_norm(N, E):
    ept = E // _NW
    scale = 1.0 - _ALPHA

    def body(packed_hbm, ew_hbm, dis_hbm, nrm_hbm, disb, pkb, wb, nvb):
        w = _wid()
        base = w * ept
        pltpu.sync_copy(dis_hbm, disb)
        pltpu.sync_copy(packed_hbm.at[pl.ds(base, ept)], pkb)
        pltpu.sync_copy(ew_hbm.at[pl.ds(base, ept)], wb)

        @pl.loop(0, ept // _L)
        def _edges(i):
            sl = pl.ds(i * _L, _L)
            pk = pkb[sl]
            r = pk & 0xFFFF
            c = pk >> 16
            dr = plsc.load_gather(disb, [r])
            dc = plsc.load_gather(disb, [c])
            nvb[sl] = (dr * wb[sl]) * (dc * scale)

        pltpu.sync_copy(nvb, nrm_hbm.at[pl.ds(base, ept)])

    return pl.kernel(
        body,
        out_type=jax.ShapeDtypeStruct((E,), jnp.float32),
        mesh=_mesh(),
        compiler_params=pltpu.CompilerParams(needs_layout_passes=False),
        scratch_types=[
            pltpu.VMEM((N,), jnp.float32),
            pltpu.VMEM((ept,), jnp.int32),
            pltpu.VMEM((ept,), jnp.float32),
            pltpu.VMEM((ept,), jnp.float32),
        ],
    )


def _make_prop(N, E, D, chunk):
    F = D // _NW
    nchunks = E // chunk
    assert nchunks % 2 == 0 and chunk % _L == 0
    groups = chunk // _L
    g_unroll = 5
    assert groups % g_unroll == 0

    def body(h0t_hbm, packed_hbm, nrm_hbm, ht_hbm,
             ha, hb, h0b, pk0, nv0, pk1, nv1, sem0, sem1):
        w = _wid()
        f0 = w * F

        for f in range(F):
            pltpu.sync_copy(h0t_hbm.at[pl.ds((f0 + f) * N, N)], ha.at[pl.ds(f * N, N)])

        @pl.loop(0, F * N // _L)
        def _scale(i):
            sl = pl.ds(i * _L, _L)
            h0b[sl] = ha[sl] * _ALPHA

        pltpu.async_copy(packed_hbm.at[pl.ds(0, chunk)], pk0, sem0)
        pltpu.async_copy(nrm_hbm.at[pl.ds(0, chunk)], nv0, sem0)

        def process_chunk(j, pkb, nvb, sem, pko, nvo, semo, src, dst):
            pltpu.make_async_copy(packed_hbm.at[pl.ds(0, chunk)], pkb, sem).wait()
            pltpu.make_async_copy(nrm_hbm.at[pl.ds(0, chunk)], nvb, sem).wait()
            jn = j + 1
            jn = jnp.where(jn == nchunks, 0, jn)
            pltpu.async_copy(packed_hbm.at[pl.ds(jn * chunk, chunk)], pko, semo)
            pltpu.async_copy(nrm_hbm.at[pl.ds(jn * chunk, chunk)], nvo, semo)

            @pl.loop(0, groups // g_unroll)
            def _grp(jj):
                for u in range(g_unroll):
                    sl = pl.ds(jj * (g_unroll * _L) + u * _L, _L)
                    pk = pkb[sl]
                    nv = nvb[sl]
                    r = pk & 0xFFFF
                    c = pk >> 16
                    for f in range(F):
                        ridx = r if f == 0 else r + (f * N)
                        cidx = c if f == 0 else c + (f * N)
                        msg = plsc.load_gather(src, [ridx]) * nv
                        plsc.addupdate_scatter(dst, [cidx], msg)

        def step(src, dst):
            @pl.loop(0, F * N // _L)
            def _init(i):
                sl = pl.ds(i * _L, _L)
                dst[sl] = h0b[sl]

            @pl.loop(0, nchunks, step=2)
            def _chunks(g):
                process_chunk(g, pk0, nv0, sem0, pk1, nv1, sem1, src, dst)
                process_chunk(g + 1, pk1, nv1, sem1, pk0, nv0, sem0, src, dst)

        @pl.loop(0, _K // 2)
        def _pair(k):
            step(ha, hb)
            step(hb, ha)

        pltpu.make_async_copy(packed_hbm.at[pl.ds(0, chunk)], pk0, sem0).wait()
        pltpu.make_async_copy(nrm_hbm.at[pl.ds(0, chunk)], nv0, sem0).wait()

        for f in range(F):
            pltpu.sync_copy(ha.at[pl.ds(f * N, N)], ht_hbm.at[pl.ds((f0 + f) * N, N)])

    return pl.kernel(
        body,
        out_type=jax.ShapeDtypeStruct((D * N,), jnp.float32),
        mesh=_mesh(),
        compiler_params=pltpu.CompilerParams(needs_layout_passes=False),
        scratch_types=[
            pltpu.VMEM((F * N,), jnp.float32),
            pltpu.VMEM((F * N,), jnp.float32),
            pltpu.VMEM((F * N,), jnp.float32),
            pltpu.VMEM((chunk,), jnp.int32),
            pltpu.VMEM((chunk,), jnp.float32),
            pltpu.VMEM((chunk,), jnp.int32),
            pltpu.VMEM((chunk,), jnp.float32),
            pltpu.SemaphoreType.DMA,
            pltpu.SemaphoreType.DMA,
        ],
    )


def _tc_transpose(ht):
    D, N = ht.shape

    def body(ht_ref, out_ref):
        out_ref[...] = ht_ref[...].T

    return pl.pallas_call(
        body,
        out_shape=jax.ShapeDtypeStruct((N, D), jnp.float32),
    )(ht)


@jax.jit
def kernel(x, edge_index, edge_weight, W):
    N, _ = x.shape
    D = W.shape[1]
    E = edge_weight.shape[0]

    parts, packed = _make_deg_pack(N, E)(edge_index[0], edge_index[1], edge_weight)
    h0t, dis = _tc_prep(x, W, parts.reshape(_NW, N))
    nrm = _make_norm(N, E)(packed, edge_weight, dis.reshape(N))
    ht = _make_prop(N, E, D, chunk=2000)(h0t.reshape(D * N), packed, nrm)
    return _tc_transpose(ht.reshape(D, N))
